# Initial kernel scaffold; baseline (speedup 1.0000x reference)
import jax
import jax.numpy as jnp
from jax.experimental import pallas as pl


def kernel(sequences, lengths, probs_x, probs_y):
    raise NotImplementedError("write your pallas kernel here")


# speedup vs baseline: 2.5811x; 2.5811x over previous
import math

import jax
import jax.numpy as jnp
from jax.experimental import pallas as pl
from jax.experimental.pallas import tpu as pltpu

_B, _T, _D, _K = 16, 512, 128, 64


def _hmm_kernel(seq_ref, len_ref, px_ref, py_ref, out_ref, ehat_ref):
    P = px_ref[...]
    py = py_ref[...]
    log_px = jnp.log(P)
    log_py = jnp.log(py)
    log_1mpy = jnp.log1p(-py)

    W = (log_py - log_1mpy).T
    bias = jnp.sum(log_1mpy, axis=1)
    seq = seq_ref[...].reshape(_T * _B, _D)
    emis = jnp.dot(seq, W, preferred_element_type=jnp.float32) + bias[None, :]

    m = jnp.max(emis, axis=1, keepdims=True)
    ehat_ref[...] = jnp.exp(emis - m).reshape(_T, _B, _K)

    lengths = len_ref[...]
    t_ids = jax.lax.broadcasted_iota(jnp.int32, (_T, _B), 0)
    mask_tb = t_ids < lengths.reshape(1, _B)
    m_sum = jnp.sum(jnp.where(mask_tb, m.reshape(_T, _B), 0.0))

    k_ids = jax.lax.broadcasted_iota(jnp.int32, (_B, _K), 1)
    alpha0 = jnp.where(k_ids == 0, 1.0, 0.0).astype(jnp.float32)
    logz0 = jnp.zeros((_B, 1), dtype=jnp.float32)

    def body(t, carry):
        alpha, logz = carry
        e = ehat_ref[t]
        a1 = jnp.dot(alpha, P, preferred_element_type=jnp.float32)
        new = a1 * e
        s = jnp.sum(new, axis=1, keepdims=True)
        alpha = new / s
        live = t < lengths
        logz = logz + jnp.where(live, jnp.log(s), 0.0)
        return alpha, logz

    _, logz = jax.lax.fori_loop(0, _T, body, (alpha0, logz0))
    loglik = jnp.sum(logz) + m_sum

    dir_const = _K * math.lgamma(1.0 + 0.1 * (_K - 1)) \
        - _K * (_K - 1) * math.lgamma(0.1)
    trace_lpx = jnp.sum(jnp.where(
        jax.lax.broadcasted_iota(jnp.int32, (_K, _K), 0)
        == jax.lax.broadcasted_iota(jnp.int32, (_K, _K), 1), log_px, 0.0))
    dir_lp = 0.9 * (trace_lpx - jnp.sum(log_px)) + dir_const
    beta_const = -_K * _D * (math.lgamma(0.1) + math.lgamma(0.9))
    beta_lp = -0.9 * jnp.sum(log_py) - 0.1 * jnp.sum(log_1mpy) + beta_const

    out_ref[0, 0] = loglik + dir_lp + beta_lp


def kernel(sequences, lengths, probs_x, probs_y):
    seq_t = jnp.swapaxes(sequences, 0, 1)
    len2d = lengths.astype(jnp.int32).reshape(_B, 1)
    out = pl.pallas_call(
        _hmm_kernel,
        out_shape=jax.ShapeDtypeStruct((1, 1), jnp.float32),
        out_specs=pl.BlockSpec(memory_space=pltpu.SMEM),
        scratch_shapes=[pltpu.VMEM((_T, _B, _K), jnp.float32)],
    )(seq_t, len2d, probs_x, probs_y)
    return out.reshape(())


# speedup vs baseline: 4.0395x; 1.5651x over previous
import math

import jax
import jax.numpy as jnp
from jax.experimental import pallas as pl
from jax.experimental.pallas import tpu as pltpu

_B, _T, _D, _K = 16, 512, 128, 64


def _hmm_kernel(seq_ref, len_ref, px_ref, py_ref, out_ref, ehat_ref):
    P = px_ref[...]
    py = py_ref[...]
    log_px = jnp.log(P)
    log_py = jnp.log(py)
    log_1mpy = jnp.log1p(-py)

    W = (log_py - log_1mpy).T
    bias = jnp.sum(log_1mpy, axis=1)
    seq = seq_ref[...].reshape(_T * _B, _D)
    emis = jnp.dot(seq, W, preferred_element_type=jnp.float32) + bias[None, :]

    m = jnp.max(emis, axis=1, keepdims=True)
    ehat_ref[...] = jnp.exp(emis - m).reshape(_T, _B, _K)

    lengths = len_ref[...]
    t_ids = jax.lax.broadcasted_iota(jnp.int32, (_T, _B), 0)
    mask_tb = t_ids < lengths.reshape(1, _B)
    m_sum = jnp.sum(jnp.where(mask_tb, m.reshape(_T, _B), 0.0))

    k_ids = jax.lax.broadcasted_iota(jnp.int32, (_B, _K), 1)
    alpha0 = jnp.where(k_ids == 0, 1.0, 0.0).astype(jnp.float32)
    logz0 = jnp.zeros((_B, 1), dtype=jnp.float32)

    r0 = jnp.ones((_B, 1), dtype=jnp.float32)

    def body(t, carry):
        u, r, logz = carry
        e = ehat_ref[t]
        v = jnp.dot(u, P, preferred_element_type=jnp.float32) * (e * r)
        s = jnp.sum(v, axis=1, keepdims=True)
        live = t < lengths
        logz = logz + jnp.where(live, jnp.log(s), 0.0)
        return v, 1.0 / s, logz

    _, _, logz = jax.lax.fori_loop(0, _T, body, (alpha0, r0, logz0),
                                   unroll=8)
    loglik = jnp.sum(logz) + m_sum

    dir_const = _K * math.lgamma(1.0 + 0.1 * (_K - 1)) \
        - _K * (_K - 1) * math.lgamma(0.1)
    trace_lpx = jnp.sum(jnp.where(
        jax.lax.broadcasted_iota(jnp.int32, (_K, _K), 0)
        == jax.lax.broadcasted_iota(jnp.int32, (_K, _K), 1), log_px, 0.0))
    dir_lp = 0.9 * (trace_lpx - jnp.sum(log_px)) + dir_const
    beta_const = -_K * _D * (math.lgamma(0.1) + math.lgamma(0.9))
    beta_lp = -0.9 * jnp.sum(log_py) - 0.1 * jnp.sum(log_1mpy) + beta_const

    out_ref[0, 0] = loglik + dir_lp + beta_lp


def kernel(sequences, lengths, probs_x, probs_y):
    seq_t = jnp.swapaxes(sequences, 0, 1)
    len2d = lengths.astype(jnp.int32).reshape(_B, 1)
    out = pl.pallas_call(
        _hmm_kernel,
        out_shape=jax.ShapeDtypeStruct((1, 1), jnp.float32),
        out_specs=pl.BlockSpec(memory_space=pltpu.SMEM),
        scratch_shapes=[pltpu.VMEM((_T, _B, _K), jnp.float32)],
    )(seq_t, len2d, probs_x, probs_y)
    return out.reshape(())


# speedup vs baseline: 4.7505x; 1.1760x over previous
import math

import jax
import jax.numpy as jnp
from jax.experimental import pallas as pl
from jax.experimental.pallas import tpu as pltpu

_B, _T, _D, _K = 16, 512, 128, 64


def _hmm_kernel(seq_ref, len_ref, px_ref, pxt_ref, py_ref, out_ref, ehat_ref):
    P = px_ref[...]
    PT = pxt_ref[...]
    py = py_ref[...]
    log_px = jnp.log(P)
    log_py = jnp.log(py)
    log_1mpy = jnp.log1p(-py)

    W = (log_py - log_1mpy).T
    bias = jnp.sum(log_1mpy, axis=1)
    seq = seq_ref[...].reshape(_T * _B, _D)
    emis = jnp.dot(seq, W, preferred_element_type=jnp.float32) + bias[None, :]

    m = jnp.max(emis, axis=1, keepdims=True)
    len_bk = len_ref[...]
    t_ids = jax.lax.broadcasted_iota(jnp.int32, (_T, _B), 0)
    mask_tb = t_ids < len_bk[:, :1].reshape(1, _B)
    m_sum = jnp.sum(jnp.where(mask_tb, m.reshape(_T, _B), 0.0))

    t3 = jax.lax.broadcasted_iota(jnp.int32, (_T, _B, _K), 0)
    mask3 = t3 < len_bk.reshape(1, _B, _K)
    ehat_ref[...] = jnp.where(
        mask3, jnp.exp(emis - m).reshape(_T, _B, _K), 1.0)

    k_ids = jax.lax.broadcasted_iota(jnp.int32, (_B, _K), 1)
    u0 = jnp.where(k_ids == 0, 1.0, 0.0).astype(jnp.float32)
    w0 = jnp.ones((_B, _K), dtype=jnp.float32)
    ones_b1 = jnp.ones((_B, 1), dtype=jnp.float32)
    zeros_b1 = jnp.zeros((_B, 1), dtype=jnp.float32)
    _H = _T // 2

    def body(i, carry):
        u, rf, logzf, w, rw, logzb = carry
        ef = ehat_ref[i]
        eb = ehat_ref[_T - 1 - i]
        v = jnp.dot(u, P, preferred_element_type=jnp.float32) * (ef * rf)
        sf = jnp.sum(v, axis=1, keepdims=True)
        logzf = logzf + jnp.log(sf)
        x = jnp.dot(w * (eb * rw), PT, preferred_element_type=jnp.float32)
        sb = jnp.sum(x, axis=1, keepdims=True)
        logzb = logzb + jnp.log(sb)
        return v, 1.0 / sf, logzf, x, _K / sb, logzb

    u, _, logzf, w, _, logzb = jax.lax.fori_loop(
        0, _H, body, (u0, ones_b1, zeros_b1, w0, ones_b1, zeros_b1),
        unroll=8)

    su = jnp.sum(u, axis=1, keepdims=True)
    sw = jnp.sum(w, axis=1, keepdims=True)
    comb = jnp.log(jnp.sum(u * w, axis=1, keepdims=True) / (su * sw))
    loglik = (jnp.sum(logzf + logzb + comb)
              + _B * (1.0 - _H) * math.log(_K) + m_sum)

    dir_const = _K * math.lgamma(1.0 + 0.1 * (_K - 1)) \
        - _K * (_K - 1) * math.lgamma(0.1)
    trace_lpx = jnp.sum(jnp.where(
        jax.lax.broadcasted_iota(jnp.int32, (_K, _K), 0)
        == jax.lax.broadcasted_iota(jnp.int32, (_K, _K), 1), log_px, 0.0))
    dir_lp = 0.9 * (trace_lpx - jnp.sum(log_px)) + dir_const
    beta_const = -_K * _D * (math.lgamma(0.1) + math.lgamma(0.9))
    beta_lp = -0.9 * jnp.sum(log_py) - 0.1 * jnp.sum(log_1mpy) + beta_const

    out_ref[0, 0] = loglik + dir_lp + beta_lp


def kernel(sequences, lengths, probs_x, probs_y):
    seq_t = jnp.swapaxes(sequences, 0, 1)
    len2d = jnp.broadcast_to(
        lengths.astype(jnp.int32).reshape(_B, 1), (_B, _K))
    out = pl.pallas_call(
        _hmm_kernel,
        out_shape=jax.ShapeDtypeStruct((1, 1), jnp.float32),
        out_specs=pl.BlockSpec(memory_space=pltpu.SMEM),
        scratch_shapes=[pltpu.VMEM((_T, _B, _K), jnp.float32)],
    )(seq_t, len2d, probs_x, probs_x.T, probs_y)
    return out.reshape(())


# speedup vs baseline: 5.1808x; 1.0906x over previous
import math

import jax
import jax.numpy as jnp
from jax.experimental import pallas as pl
from jax.experimental.pallas import tpu as pltpu

_B, _T, _D, _K = 16, 512, 128, 64


def _hmm_kernel(seq_ref, len_ref, px_ref, pxt_ref, py_ref, out_ref, ehat_ref):
    P = px_ref[...]
    PT = pxt_ref[...]
    py = py_ref[...]
    log_px = jnp.log(P)
    log_py = jnp.log(py)
    log_1mpy = jnp.log1p(-py)

    W = (log_py - log_1mpy).T
    bias = jnp.sum(log_1mpy, axis=1)
    seq = seq_ref[...].reshape(_T * _B, _D)
    emis = jnp.dot(seq, W, preferred_element_type=jnp.float32) + bias[None, :]

    m = jnp.max(emis, axis=1, keepdims=True)
    len_bk = len_ref[...]
    t_ids = jax.lax.broadcasted_iota(jnp.int32, (_T, _B), 0)
    mask_tb = t_ids < len_bk[:, :1].reshape(1, _B)
    m_sum = jnp.sum(jnp.where(mask_tb, m.reshape(_T, _B), 0.0))

    t3 = jax.lax.broadcasted_iota(jnp.int32, (_T, _B, _K), 0)
    mask3 = t3 < len_bk.reshape(1, _B, _K)
    ehat_ref[...] = jnp.where(
        mask3, jnp.exp(emis - m).reshape(_T, _B, _K), 1.0)

    k_ids = jax.lax.broadcasted_iota(jnp.int32, (_B, _K), 1)
    u0 = jnp.where(k_ids == 0, 1.0, 0.0).astype(jnp.float32)
    w0 = jnp.ones((_B, _K), dtype=jnp.float32)
    ones_b1 = jnp.ones((_B, 1), dtype=jnp.float32)
    zeros_b1 = jnp.zeros((_B, 1), dtype=jnp.float32)
    _H = _T // 2

    def body(i, carry):
        u, rf, logzf, w, rw, logzb = carry
        t0 = 4 * i
        v = jnp.dot(u, P, preferred_element_type=jnp.float32) \
            * (ehat_ref[t0] * rf)
        for q in range(1, 4):
            v = jnp.dot(v, P, preferred_element_type=jnp.float32) \
                * ehat_ref[t0 + q]
        sf = jnp.sum(v, axis=1, keepdims=True)
        logzf = logzf + jnp.log(sf)
        x = jnp.dot(w * (ehat_ref[_T - 1 - t0] * rw), PT,
                    preferred_element_type=jnp.float32)
        for q in range(1, 4):
            x = jnp.dot(x * ehat_ref[_T - 1 - t0 - q], PT,
                        preferred_element_type=jnp.float32)
        sb = jnp.sum(x, axis=1, keepdims=True)
        logzb = logzb + jnp.log(sb)
        return v, 1.0 / sf, logzf, x, _K / sb, logzb

    u, _, logzf, w, _, logzb = jax.lax.fori_loop(
        0, _H // 4, body, (u0, ones_b1, zeros_b1, w0, ones_b1, zeros_b1),
        unroll=2)

    su = jnp.sum(u, axis=1, keepdims=True)
    sw = jnp.sum(w, axis=1, keepdims=True)
    comb = jnp.log(jnp.sum(u * w, axis=1, keepdims=True) / (su * sw))
    loglik = (jnp.sum(logzf + logzb + comb)
              + _B * (1.0 - _H / 4) * math.log(_K) + m_sum)

    dir_const = _K * math.lgamma(1.0 + 0.1 * (_K - 1)) \
        - _K * (_K - 1) * math.lgamma(0.1)
    trace_lpx = jnp.sum(jnp.where(
        jax.lax.broadcasted_iota(jnp.int32, (_K, _K), 0)
        == jax.lax.broadcasted_iota(jnp.int32, (_K, _K), 1), log_px, 0.0))
    dir_lp = 0.9 * (trace_lpx - jnp.sum(log_px)) + dir_const
    beta_const = -_K * _D * (math.lgamma(0.1) + math.lgamma(0.9))
    beta_lp = -0.9 * jnp.sum(log_py) - 0.1 * jnp.sum(log_1mpy) + beta_const

    out_ref[0, 0] = loglik + dir_lp + beta_lp


def kernel(sequences, lengths, probs_x, probs_y):
    seq_t = jnp.swapaxes(sequences, 0, 1)
    len2d = jnp.broadcast_to(
        lengths.astype(jnp.int32).reshape(_B, 1), (_B, _K))
    out = pl.pallas_call(
        _hmm_kernel,
        out_shape=jax.ShapeDtypeStruct((1, 1), jnp.float32),
        out_specs=pl.BlockSpec(memory_space=pltpu.SMEM),
        scratch_shapes=[pltpu.VMEM((_T, _B, _K), jnp.float32)],
    )(seq_t, len2d, probs_x, probs_x.T, probs_y)
    return out.reshape(())
